# baseline (speedup 1.0000x reference)
import jax
import jax.numpy as jnp
from jax import lax
from jax.experimental import pallas as pl
from jax.experimental.pallas import tpu as pltpu
from jax.experimental.pallas import tpu_sc as plsc

N_NODES = 10000
D_FEAT = 128
N_EDGES = 320000
NC = 2
NS = 16
NW = NC * NS
EPW = N_EDGES // NW
K = 128
NCHUNK = 80
HALF = NCHUNK // 2
PAIRS_PER_HALF = HALF // 2
EPW_PAD = NCHUNK * K
ACC_ROWS = N_NODES + 8
ROWS_PER_TILE = (N_NODES // NS) // 8 * 8
ROWS_TAIL = N_NODES - NS * ROWS_PER_TILE


def _sc_partial_body(x_hbm, src_hbm, dst_hbm, p_hbm,
                     src_v, dst_v, rows0, rows1,
                     acc, sem0, sem1):
    cid = lax.axis_index("c")
    sid = lax.axis_index("s")
    wid = cid * NS + sid

    r0 = sid * ROWS_PER_TILE
    pltpu.sync_copy(x_hbm.at[pl.ds(r0, ROWS_PER_TILE)],
                    acc.at[pl.ds(r0, ROWS_PER_TILE)])

    @pl.when(sid == NS - 1)
    def _init_tail():
        t0 = NS * ROWS_PER_TILE
        pltpu.sync_copy(x_hbm.at[pl.ds(t0, ROWS_TAIL)],
                        acc.at[pl.ds(t0, ROWS_TAIL)])

    plsc.subcore_barrier()

    for h in (0, 1):
        pltpu.sync_copy(src_hbm.at[wid, pl.ds(h * HALF, HALF)], src_v)
        pltpu.sync_copy(dst_hbm.at[wid, pl.ds(h * HALF, HALF)], dst_v)

        def chunk(j, carry):
            pltpu.async_copy(x_hbm.at[src_v.at[j]], rows0, sem0).wait()
            pltpu.sync_copy(rows0, acc.at[dst_v.at[j]], add=True)
            return carry

        lax.fori_loop(0, HALF, chunk, 0)

    plsc.subcore_barrier()
    pltpu.sync_copy(acc.at[pl.ds(r0, ROWS_PER_TILE)],
                    p_hbm.at[cid, pl.ds(r0, ROWS_PER_TILE)])

    @pl.when(sid == NS - 1)
    def _drain_tail():
        t0 = NS * ROWS_PER_TILE
        pltpu.sync_copy(acc.at[pl.ds(t0, ROWS_TAIL)],
                        p_hbm.at[cid, pl.ds(t0, ROWS_TAIL)])


def _combine_body(x_ref, p_ref, o_ref):
    o_ref[...] = p_ref[0] + p_ref[1] - x_ref[...]


def kernel(x, edge_index):
    src = edge_index[0].astype(jnp.int32).reshape(NW, EPW)
    dst = edge_index[1].astype(jnp.int32).reshape(NW, EPW)
    pad = EPW_PAD - EPW
    src3 = jnp.pad(src, ((0, 0), (0, pad))).reshape(NW, NCHUNK, K)
    dst3 = jnp.pad(dst, ((0, 0), (0, pad)),
                   constant_values=N_NODES).reshape(NW, NCHUNK, K)

    mesh = plsc.VectorSubcoreMesh(core_axis_name="c", subcore_axis_name="s",
                                  num_cores=NC, num_subcores=NS)
    p = pl.kernel(
        _sc_partial_body,
        out_type=jax.ShapeDtypeStruct((NC, N_NODES, D_FEAT), jnp.float32),
        mesh=mesh,
        scratch_types=[
            pltpu.VMEM((HALF, K), jnp.int32),
            pltpu.VMEM((HALF, K), jnp.int32),
            pltpu.VMEM((K, D_FEAT), jnp.float32),
            pltpu.VMEM((K, D_FEAT), jnp.float32),
            pltpu.VMEM_SHARED((ACC_ROWS, D_FEAT), jnp.float32),
            pltpu.SemaphoreType.DMA,
            pltpu.SemaphoreType.DMA,
        ],
    )(x, src3, dst3)

    BLK = 400
    out = pl.pallas_call(
        _combine_body,
        out_shape=jax.ShapeDtypeStruct((N_NODES, D_FEAT), jnp.float32),
        grid=(N_NODES // BLK,),
        in_specs=[pl.BlockSpec((BLK, D_FEAT), lambda i: (i, 0)),
                  pl.BlockSpec((NC, BLK, D_FEAT), lambda i: (0, i, 0))],
        out_specs=pl.BlockSpec((BLK, D_FEAT), lambda i: (i, 0)),
    )(x, p)
    return out


# speedup vs baseline: 2.5920x; 2.5920x over previous
import jax
import jax.numpy as jnp
from jax import lax
from jax.experimental import pallas as pl
from jax.experimental.pallas import tpu as pltpu
from jax.experimental.pallas import tpu_sc as plsc

N_NODES = 10000
D_FEAT = 128
N_EDGES = 320000
NC = 2
NS = 16
NW = NC * NS
EPW = N_EDGES // NW
K = 128
NFULL = EPW // K
NPAIR = NFULL // 2
REM = EPW - NFULL * K
ROWS_PER_TILE = (N_NODES // NS) // 8 * 8
ROWS_TAIL = N_NODES - NS * ROWS_PER_TILE


def _sc_partial_body(x_hbm, src_hbm, dst_hbm, p_hbm,
                     srcA, dstA, srcB, dstB, rowsA, rowsB,
                     srcr, dstr, rowsr,
                     acc, semA, semB):
    cid = lax.axis_index("c")
    sid = lax.axis_index("s")
    wid = cid * NS + sid
    base = wid * EPW

    r0 = sid * ROWS_PER_TILE
    pltpu.sync_copy(x_hbm.at[pl.ds(r0, ROWS_PER_TILE)],
                    acc.at[pl.ds(r0, ROWS_PER_TILE)])

    @pl.when(sid == NS - 1)
    def _init_tail():
        t0 = NS * ROWS_PER_TILE
        pltpu.sync_copy(x_hbm.at[pl.ds(t0, ROWS_TAIL)],
                        acc.at[pl.ds(t0, ROWS_TAIL)])

    plsc.subcore_barrier()

    pltpu.sync_copy(src_hbm.at[pl.ds(base, K)], srcA)
    pltpu.sync_copy(dst_hbm.at[pl.ds(base, K)], dstA)
    pltpu.async_copy(x_hbm.at[srcA], rowsA, semA)

    def pair(i, carry):
        j0 = 2 * i
        off1 = base + (j0 + 1) * K
        pltpu.sync_copy(src_hbm.at[pl.ds(off1, K)], srcB)
        pltpu.sync_copy(dst_hbm.at[pl.ds(off1, K)], dstB)
        pltpu.async_copy(x_hbm.at[srcB], rowsB, semB)
        pltpu.make_async_copy(x_hbm.at[srcA], rowsA, semA).wait()
        pltpu.sync_copy(rowsA, acc.at[dstA], add=True)

        @pl.when(i < NPAIR - 1)
        def _next():
            off2 = base + (j0 + 2) * K
            pltpu.sync_copy(src_hbm.at[pl.ds(off2, K)], srcA)
            pltpu.sync_copy(dst_hbm.at[pl.ds(off2, K)], dstA)
            pltpu.async_copy(x_hbm.at[srcA], rowsA, semA)

        pltpu.make_async_copy(x_hbm.at[srcB], rowsB, semB).wait()
        pltpu.sync_copy(rowsB, acc.at[dstB], add=True)
        return carry

    lax.fori_loop(0, NPAIR, pair, 0)

    if REM:
        off = base + NFULL * K
        pltpu.sync_copy(src_hbm.at[pl.ds(off, REM)], srcr)
        pltpu.sync_copy(dst_hbm.at[pl.ds(off, REM)], dstr)
        pltpu.async_copy(x_hbm.at[srcr], rowsr, semA).wait()
        pltpu.sync_copy(rowsr, acc.at[dstr], add=True)

    plsc.subcore_barrier()
    pltpu.sync_copy(acc.at[pl.ds(r0, ROWS_PER_TILE)],
                    p_hbm.at[cid, pl.ds(r0, ROWS_PER_TILE)])

    @pl.when(sid == NS - 1)
    def _drain_tail():
        t0 = NS * ROWS_PER_TILE
        pltpu.sync_copy(acc.at[pl.ds(t0, ROWS_TAIL)],
                        p_hbm.at[cid, pl.ds(t0, ROWS_TAIL)])


def _combine_body(x_ref, p_ref, o_ref):
    o_ref[...] = p_ref[0] + p_ref[1] - x_ref[...]


def kernel(x, edge_index):
    src = edge_index[0].astype(jnp.int32)
    dst = edge_index[1].astype(jnp.int32)

    mesh = plsc.VectorSubcoreMesh(core_axis_name="c", subcore_axis_name="s",
                                  num_cores=NC, num_subcores=NS)
    p = pl.kernel(
        _sc_partial_body,
        out_type=jax.ShapeDtypeStruct((NC, N_NODES, D_FEAT), jnp.float32),
        mesh=mesh,
        scratch_types=[
            pltpu.VMEM((K,), jnp.int32),
            pltpu.VMEM((K,), jnp.int32),
            pltpu.VMEM((K,), jnp.int32),
            pltpu.VMEM((K,), jnp.int32),
            pltpu.VMEM((K, D_FEAT), jnp.float32),
            pltpu.VMEM((K, D_FEAT), jnp.float32),
            pltpu.VMEM((REM,), jnp.int32),
            pltpu.VMEM((REM,), jnp.int32),
            pltpu.VMEM((REM, D_FEAT), jnp.float32),
            pltpu.VMEM_SHARED((N_NODES, D_FEAT), jnp.float32),
            pltpu.SemaphoreType.DMA,
            pltpu.SemaphoreType.DMA,
        ],
    )(x, src, dst)

    BLK = 400
    out = pl.pallas_call(
        _combine_body,
        out_shape=jax.ShapeDtypeStruct((N_NODES, D_FEAT), jnp.float32),
        grid=(N_NODES // BLK,),
        in_specs=[pl.BlockSpec((BLK, D_FEAT), lambda i: (i, 0)),
                  pl.BlockSpec((NC, BLK, D_FEAT), lambda i: (0, i, 0))],
        out_specs=pl.BlockSpec((BLK, D_FEAT), lambda i: (i, 0)),
    )(x, p)
    return out


# speedup vs baseline: 2.7589x; 1.0644x over previous
import jax
import jax.numpy as jnp
from jax import lax
from jax.experimental import pallas as pl
from jax.experimental.pallas import tpu as pltpu
from jax.experimental.pallas import tpu_sc as plsc

N_NODES = 10000
D_FEAT = 128
N_EDGES = 320000
NC = 2
NS = 16
NW = NC * NS
EPW = N_EDGES // NW
K = 96
R = 4
NFULL = EPW // K
T = NFULL // R
REM = EPW - NFULL * K
ROWS_PER_TILE = (N_NODES // NS) // 8 * 8
ROWS_TAIL = N_NODES - NS * ROWS_PER_TILE


def _sc_partial_body(x_hbm, src_hbm, dst_hbm, p_hbm,
                     src0, src1, src2, src3,
                     dst0, dst1, dst2, dst3,
                     rows0, rows1, rows2, rows3,
                     srcr, dstr,
                     acc,
                     isem0, isem1, isem2, isem3,
                     gsem0, gsem1, gsem2, gsem3,
                     ssem0, ssem1, ssem2, ssem3):
    cid = lax.axis_index("c")
    sid = lax.axis_index("s")
    wid = cid * NS + sid
    base = wid * EPW

    srcs = (src0, src1, src2, src3)
    dsts = (dst0, dst1, dst2, dst3)
    rows = (rows0, rows1, rows2, rows3)
    isems = (isem0, isem1, isem2, isem3)
    gsems = (gsem0, gsem1, gsem2, gsem3)
    ssems = (ssem0, ssem1, ssem2, ssem3)

    def idx_fire(k, off):
        pltpu.async_copy(src_hbm.at[pl.ds(off, K)], srcs[k], isems[k])
        pltpu.async_copy(dst_hbm.at[pl.ds(off, K)], dsts[k], isems[k])

    def idx_wait(k, off):
        pltpu.make_async_copy(src_hbm.at[pl.ds(off, K)], srcs[k], isems[k]).wait()
        pltpu.make_async_copy(dst_hbm.at[pl.ds(off, K)], dsts[k], isems[k]).wait()

    def gather_fire(k):
        pltpu.async_copy(x_hbm.at[srcs[k]], rows[k], gsems[k])

    def gather_wait(k):
        pltpu.make_async_copy(x_hbm.at[srcs[k]], rows[k], gsems[k]).wait()

    def scatter_fire(k):
        pltpu.async_copy(rows[k], acc.at[dsts[k]], ssems[k], add=True)

    def scatter_wait(k):
        pltpu.make_async_copy(rows[k], acc.at[dsts[k]], ssems[k]).wait()

    r0 = sid * ROWS_PER_TILE
    pltpu.sync_copy(x_hbm.at[pl.ds(r0, ROWS_PER_TILE)],
                    acc.at[pl.ds(r0, ROWS_PER_TILE)])

    @pl.when(sid == NS - 1)
    def _init_tail():
        t0 = NS * ROWS_PER_TILE
        pltpu.sync_copy(x_hbm.at[pl.ds(t0, ROWS_TAIL)],
                        acc.at[pl.ds(t0, ROWS_TAIL)])

    plsc.subcore_barrier()

    for k in range(R):
        idx_fire(k, base + k * K)
    for k in range(R):
        idx_wait(k, base + k * K)
        gather_fire(k)

    def round_(t, carry):
        for k in range(R):
            gather_wait(k)
            scatter_fire(k)

        @pl.when(t < T - 1)
        def _restage():
            for k in range(R):
                off = base + (R * t + R + k) * K
                scatter_wait(k)
                idx_fire(k, off)
            for k in range(R):
                off = base + (R * t + R + k) * K
                idx_wait(k, off)
                gather_fire(k)

        return carry

    lax.fori_loop(0, T, round_, 0)

    for k in range(R):
        scatter_wait(k)

    if REM:
        off = base + NFULL * K
        pltpu.sync_copy(src_hbm.at[pl.ds(off, REM)], srcr)
        pltpu.sync_copy(dst_hbm.at[pl.ds(off, REM)], dstr)
        pltpu.async_copy(x_hbm.at[srcr], rows0.at[pl.ds(0, REM)], gsem0).wait()
        pltpu.sync_copy(rows0.at[pl.ds(0, REM)], acc.at[dstr], add=True)

    plsc.subcore_barrier()
    pltpu.sync_copy(acc.at[pl.ds(r0, ROWS_PER_TILE)],
                    p_hbm.at[cid, pl.ds(r0, ROWS_PER_TILE)])

    @pl.when(sid == NS - 1)
    def _drain_tail():
        t0 = NS * ROWS_PER_TILE
        pltpu.sync_copy(acc.at[pl.ds(t0, ROWS_TAIL)],
                        p_hbm.at[cid, pl.ds(t0, ROWS_TAIL)])


def _combine_body(x_ref, p_ref, o_ref):
    o_ref[...] = p_ref[0] + p_ref[1] - x_ref[...]


def kernel(x, edge_index):
    src = edge_index[0].astype(jnp.int32)
    dst = edge_index[1].astype(jnp.int32)

    mesh = plsc.VectorSubcoreMesh(core_axis_name="c", subcore_axis_name="s",
                                  num_cores=NC, num_subcores=NS)
    p = pl.kernel(
        _sc_partial_body,
        out_type=jax.ShapeDtypeStruct((NC, N_NODES, D_FEAT), jnp.float32),
        mesh=mesh,
        scratch_types=(
            [pltpu.VMEM((K,), jnp.int32) for _ in range(2 * R)]
            + [pltpu.VMEM((K, D_FEAT), jnp.float32) for _ in range(R)]
            + [pltpu.VMEM((REM,), jnp.int32) for _ in range(2)]
            + [pltpu.VMEM_SHARED((N_NODES, D_FEAT), jnp.float32)]
            + [pltpu.SemaphoreType.DMA for _ in range(3 * R)]
        ),
    )(x, src, dst)

    BLK = 400
    out = pl.pallas_call(
        _combine_body,
        out_shape=jax.ShapeDtypeStruct((N_NODES, D_FEAT), jnp.float32),
        grid=(N_NODES // BLK,),
        in_specs=[pl.BlockSpec((BLK, D_FEAT), lambda i: (i, 0)),
                  pl.BlockSpec((NC, BLK, D_FEAT), lambda i: (0, i, 0))],
        out_specs=pl.BlockSpec((BLK, D_FEAT), lambda i: (i, 0)),
    )(x, p)
    return out


# speedup vs baseline: 3.1457x; 1.1402x over previous
import jax
import jax.numpy as jnp
from jax import lax
from jax.experimental import pallas as pl
from jax.experimental.pallas import tpu as pltpu
from jax.experimental.pallas import tpu_sc as plsc

N_NODES = 10000
D_FEAT = 128
N_EDGES = 320000
NC = 2
NS = 16
NW = NC * NS
EPW = N_EDGES // NW
K = 80
R = 4
SL = 2 * R
NFULL = EPW // K
T = NFULL // SL
NEPI = NFULL - T * SL
ROWS_PER_TILE = (N_NODES // NS) // 8 * 8
ROWS_TAIL = N_NODES - NS * ROWS_PER_TILE


def _sc_partial_body(x_hbm, src_hbm, dst_hbm, p_hbm, *refs):
    srcs = refs[0:SL]
    dsts = refs[SL:2 * SL]
    rows = refs[2 * SL:2 * SL + R]
    acc = refs[2 * SL + R]
    sems = refs[2 * SL + R + 1:]
    isems = sems[0:SL]
    gsems = sems[SL:SL + R]
    ssems = sems[SL + R:SL + 2 * R]

    cid = lax.axis_index("c")
    sid = lax.axis_index("s")
    wid = cid * NS + sid
    base = wid * EPW

    def idx_fire(s, off):
        pltpu.async_copy(src_hbm.at[pl.ds(off, K)], srcs[s], isems[s])
        pltpu.async_copy(dst_hbm.at[pl.ds(off, K)], dsts[s], isems[s])

    def idx_wait(s, off):
        pltpu.make_async_copy(src_hbm.at[pl.ds(off, K)], srcs[s], isems[s]).wait()
        pltpu.make_async_copy(dst_hbm.at[pl.ds(off, K)], dsts[s], isems[s]).wait()

    def gather_fire(k, s):
        pltpu.async_copy(x_hbm.at[srcs[s]], rows[k], gsems[k])

    def gather_wait(k, s):
        pltpu.make_async_copy(x_hbm.at[srcs[s]], rows[k], gsems[k]).wait()

    def scatter_fire(k, s):
        pltpu.async_copy(rows[k], acc.at[dsts[s]], ssems[k], add=True)

    def scatter_wait(k, s):
        pltpu.make_async_copy(rows[k], acc.at[dsts[s]], ssems[k]).wait()

    r0 = sid * ROWS_PER_TILE
    pltpu.sync_copy(x_hbm.at[pl.ds(r0, ROWS_PER_TILE)],
                    acc.at[pl.ds(r0, ROWS_PER_TILE)])

    @pl.when(sid == NS - 1)
    def _init_tail():
        t0 = NS * ROWS_PER_TILE
        pltpu.sync_copy(x_hbm.at[pl.ds(t0, ROWS_TAIL)],
                        acc.at[pl.ds(t0, ROWS_TAIL)])

    plsc.subcore_barrier()

    for s in range(SL):
        idx_fire(s, base + s * K)
    for k in range(R):
        idx_wait(k, base + k * K)
        gather_fire(k, k)

    def body(t, carry):
        c0 = base + (t * SL) * K

        for m in range(R):
            gather_wait(m, m)
            scatter_fire(m, m)
        for m in range(R):
            scatter_wait(m, m)
            idx_wait(m + R, c0 + (m + R) * K)
            gather_fire(m, m + R)

            @pl.when(t < T - 1)
            def _restage_lo():
                idx_fire(m, c0 + (m + SL) * K)

        for m in range(R):
            gather_wait(m, m + R)
            scatter_fire(m, m + R)
        for m in range(R):
            scatter_wait(m, m + R)

            @pl.when(t < T - 1)
            def _next_hi():
                idx_wait(m, c0 + (m + SL) * K)
                gather_fire(m, m)
                idx_fire(m + R, c0 + (m + R + SL) * K)

        return carry

    lax.fori_loop(0, T, body, 0)

    e0 = base + T * SL * K
    for s in range(NEPI):
        idx_fire(s, e0 + s * K)
    for s in range(R):
        idx_wait(s, e0 + s * K)
        gather_fire(s, s)
    idx_wait(R, e0 + R * K)
    for s in range(R):
        gather_wait(s, s)
        scatter_fire(s, s)
    scatter_wait(0, 0)
    gather_fire(0, R)
    gather_wait(0, R)
    scatter_fire(0, R)
    for s in range(1, R):
        scatter_wait(s, s)
    scatter_wait(0, R)

    plsc.subcore_barrier()
    pltpu.sync_copy(acc.at[pl.ds(r0, ROWS_PER_TILE)],
                    p_hbm.at[cid, pl.ds(r0, ROWS_PER_TILE)])

    @pl.when(sid == NS - 1)
    def _drain_tail():
        t0 = NS * ROWS_PER_TILE
        pltpu.sync_copy(acc.at[pl.ds(t0, ROWS_TAIL)],
                        p_hbm.at[cid, pl.ds(t0, ROWS_TAIL)])


def _combine_body(x_ref, p_ref, o_ref):
    o_ref[...] = p_ref[0] + p_ref[1] - x_ref[...]


def kernel(x, edge_index):
    src = edge_index[0].astype(jnp.int32)
    dst = edge_index[1].astype(jnp.int32)

    mesh = plsc.VectorSubcoreMesh(core_axis_name="c", subcore_axis_name="s",
                                  num_cores=NC, num_subcores=NS)
    p = pl.kernel(
        _sc_partial_body,
        out_type=jax.ShapeDtypeStruct((NC, N_NODES, D_FEAT), jnp.float32),
        mesh=mesh,
        scratch_types=(
            [pltpu.VMEM((K,), jnp.int32) for _ in range(2 * SL)]
            + [pltpu.VMEM((K, D_FEAT), jnp.float32) for _ in range(R)]
            + [pltpu.VMEM_SHARED((N_NODES, D_FEAT), jnp.float32)]
            + [pltpu.SemaphoreType.DMA for _ in range(SL + 2 * R)]
        ),
    )(x, src, dst)

    BLK = 400
    out = pl.pallas_call(
        _combine_body,
        out_shape=jax.ShapeDtypeStruct((N_NODES, D_FEAT), jnp.float32),
        grid=(N_NODES // BLK,),
        in_specs=[pl.BlockSpec((BLK, D_FEAT), lambda i: (i, 0)),
                  pl.BlockSpec((NC, BLK, D_FEAT), lambda i: (0, i, 0))],
        out_specs=pl.BlockSpec((BLK, D_FEAT), lambda i: (i, 0)),
    )(x, p)
    return out
